# baseline (speedup 1.0000x reference)
import jax
import jax.numpy as jnp
from jax import lax
from jax.experimental import pallas as pl
from jax.experimental.pallas import tpu as pltpu
from jax.experimental.pallas import tpu_sc as plsc

NROWS = 4096
NCOLS = 4096
NNZ = 1638400
NW = 32
SPLITS = 2
CSPLIT = NCOLS // SPLITS
COLS_PER_W = CSPLIT // NW
SLAB = 8
SLABS = COLS_PER_W // SLAB
TILE = 2048
L = 16
UNROLL = 4
TBLK = 1024


def _make_sc_body(split):
    def body(ccol_hbm, ccols_hbm, rows_hbm, vals_hbm, outt_hbm, acc0, acc1,
             rbuf0, rbuf1, vbuf0, vbuf1, bbuf, bsbuf,
             sem0, sem1, fsem0, fsem1):
        return _sc_work(split, ccol_hbm, ccols_hbm, rows_hbm, vals_hbm,
                        outt_hbm, acc0, acc1, rbuf0, rbuf1, vbuf0, vbuf1,
                        bbuf, bsbuf, sem0, sem1, fsem0, fsem1)
    return body


def _sc_work(split, ccol_hbm, ccols_hbm, rows_hbm, vals_hbm, outt_hbm,
             acc0, acc1, rbuf0, rbuf1, vbuf0, vbuf1, bbuf, bsbuf,
             sem0, sem1, fsem0, fsem1):
    wid = lax.axis_index("s") * 2 + lax.axis_index("c")
    iota = jax.lax.iota(jnp.int32, L)
    zeros16 = jnp.zeros((L,), jnp.float32)
    accs = (acc0, acc1)
    rbufs = (rbuf0, rbuf1)
    vbufs = (vbuf0, vbuf1)
    sems = (sem0, sem1)
    fsems = (fsem0, fsem1)

    gw0 = split * CSPLIT + wid * COLS_PER_W
    cw0 = wid * COLS_PER_W
    pltpu.sync_copy(ccol_hbm.at[pl.ds(gw0, COLS_PER_W + L)], bbuf)
    pltpu.sync_copy(ccols_hbm.at[pl.ds(gw0, COLS_PER_W + L)], bsbuf)

    def run_slab(ci, half, first):
        acc = accs[half]
        fsem = fsems[half]
        c0 = cw0 + ci * SLAB
        b_lo = bbuf[pl.ds(ci * SLAB, L)]
        b_hi = bsbuf[pl.ds(ci * SLAB, L)]
        start = jnp.sum(jnp.where(iota == 0, b_lo, 0))
        end = jnp.sum(jnp.where(iota == SLAB - 1, b_hi, 0))

        if not first:
            pc0 = c0 - 2 * SLAB
            pltpu.make_async_copy(
                acc, outt_hbm.at[pl.ds(pc0 * NROWS, SLAB * NROWS)],
                fsem).wait()

        base0 = start & jnp.int32(-8)
        ntiles = (end - base0 + (TILE - 1)) // TILE

        def window(t):
            return pl.multiple_of(
                jnp.minimum(base0 + t * TILE, NNZ - TILE), 8)

        def issue(t, sub):
            tb = window(t)
            pltpu.async_copy(rows_hbm.at[pl.ds(tb, TILE)], rbufs[sub],
                             sems[sub])
            pltpu.async_copy(vals_hbm.at[pl.ds(tb, TILE)], vbufs[sub],
                             sems[sub])

        def wait(t, sub):
            tb = window(t)
            pltpu.make_async_copy(rows_hbm.at[pl.ds(tb, TILE)], rbufs[sub],
                                  sems[sub]).wait()
            pltpu.make_async_copy(vals_hbm.at[pl.ds(tb, TILE)], vbufs[sub],
                                  sems[sub]).wait()

        for sub in range(2):
            @pl.when(sub < ntiles)
            def _():
                issue(sub, sub)

        def zero_body(k, _):
            for u in range(16):
                acc[pl.ds((k * 16 + u) * L, L)] = zeros16
            return 0

        lax.fori_loop(0, SLAB * NROWS // L // 16, zero_body, 0)

        def pair_body(p, _):
            for sub in range(2):
                t = p * 2 + sub

                @pl.when(t < ntiles)
                def _():
                    tb = window(t)
                    lo = jnp.maximum(start, base0 + t * TILE)
                    hi = jnp.minimum(end, base0 + (t + 1) * TILE)
                    wait(t, sub)

                    @plsc.parallel_loop(0, TILE // L, unroll=UNROLL)
                    def _(v):
                        off = v * L
                        g = tb + off + iota
                        r16 = rbufs[sub][pl.ds(off, L)]
                        vl = vbufs[sub][pl.ds(off, L)]
                        msk = (g >= lo) & (g < hi)
                        rel = jnp.zeros((L,), jnp.int32)
                        for k in (4, 2, 1):
                            probe = rel + (ci * SLAB + (k - 1))
                            b = plsc.load_gather(bsbuf, [probe])
                            rel = rel + jnp.where(g >= b, k, 0)
                        idx = jnp.left_shift(rel, 12) + r16
                        plsc.addupdate_scatter(acc, [idx], vl, mask=msk)

                    @pl.when(t + 2 < ntiles)
                    def _():
                        issue(t + 2, sub)
            return 0

        lax.fori_loop(0, (ntiles + 1) // 2, pair_body, 0)

        pltpu.async_copy(acc, outt_hbm.at[pl.ds(c0 * NROWS, SLAB * NROWS)],
                         fsem)

    def slab_pair(p, _):
        for half in range(2):
            run_slab(p * 2 + half, half, first=False)
        return 0

    for half in range(2):
        run_slab(half, half, first=True)
    lax.fori_loop(1, SLABS // 2, slab_pair, 0)

    for half in range(2):
        c0 = cw0 + (SLABS - 2 + half) * SLAB
        pltpu.make_async_copy(
            accs[half], outt_hbm.at[pl.ds(c0 * NROWS, SLAB * NROWS)],
            fsems[half]).wait()


def _transpose_body(int_ref, out_ref):
    out_ref[...] = int_ref[...].T


def _transpose_body_inplace(int_ref, din_ref, out_ref):
    del din_ref
    out_ref[...] = int_ref[...].T


def _run_sc_split(split, ccol_p, ccols_p, rows_p, vals_p):
    mesh = plsc.VectorSubcoreMesh(core_axis_name="c", subcore_axis_name="s")
    return pl.kernel(
        _make_sc_body(split),
        out_type=jax.ShapeDtypeStruct((CSPLIT * NROWS,), jnp.float32),
        mesh=mesh,
        scratch_types=[
            pltpu.VMEM((SLAB * NROWS,), jnp.float32),
            pltpu.VMEM((SLAB * NROWS,), jnp.float32),
            pltpu.VMEM((TILE,), jnp.int32),
            pltpu.VMEM((TILE,), jnp.int32),
            pltpu.VMEM((TILE,), jnp.float32),
            pltpu.VMEM((TILE,), jnp.float32),
            pltpu.VMEM((COLS_PER_W + L,), jnp.int32),
            pltpu.VMEM((COLS_PER_W + L,), jnp.int32),
            pltpu.SemaphoreType.DMA,
            pltpu.SemaphoreType.DMA,
            pltpu.SemaphoreType.DMA,
            pltpu.SemaphoreType.DMA,
        ],
        compiler_params=pltpu.CompilerParams(needs_layout_passes=False),
    )(ccol_p, ccols_p, rows_p, vals_p)


def _run_tc_split(split, outt, dense_in):
    grid = (NROWS // TBLK, CSPLIT // TBLK)
    cofs = split * (CSPLIT // TBLK)
    if dense_in is None:
        return pl.pallas_call(
            _transpose_body,
            grid=grid,
            in_specs=[pl.BlockSpec((TBLK, TBLK), lambda i, j: (j, i))],
            out_specs=pl.BlockSpec((TBLK, TBLK),
                                   lambda i, j: (i, j + cofs)),
            out_shape=jax.ShapeDtypeStruct((NROWS, NCOLS), jnp.float32),
        )(outt)
    return pl.pallas_call(
        _transpose_body_inplace,
        grid=grid,
        in_specs=[
            pl.BlockSpec((TBLK, TBLK), lambda i, j: (j, i)),
            pl.BlockSpec(memory_space=pltpu.HBM),
        ],
        out_specs=pl.BlockSpec((TBLK, TBLK), lambda i, j: (i, j + cofs)),
        out_shape=jax.ShapeDtypeStruct((NROWS, NCOLS), jnp.float32),
        input_output_aliases={1: 0},
    )(outt, dense_in)


@jax.jit
def _csc_to_dense(ccol_p, ccols_p, rows_p, vals_p):
    outts = [_run_sc_split(s, ccol_p, ccols_p, rows_p, vals_p)
             for s in range(SPLITS)]
    dense = None
    for s in range(SPLITS):
        dense = _run_tc_split(s, outts[s].reshape(CSPLIT, NROWS), dense)
    return dense


def kernel(ccol_indices, row_indices, values):
    ccol = ccol_indices.astype(jnp.int32)
    rows = row_indices.astype(jnp.int32)
    vals = values.astype(jnp.float32)
    ccol_p = jnp.pad(ccol, (0, 4256 - ccol.shape[0]))
    ccols_p = jnp.pad(ccol[1:], (0, 4256 - ccol.shape[0] + 1))
    return _csc_to_dense(ccol_p, ccols_p, rows, vals)


# speedup vs baseline: 1.3110x; 1.3110x over previous
import jax
import jax.numpy as jnp
from jax import lax
from jax.experimental import pallas as pl
from jax.experimental.pallas import tpu as pltpu
from jax.experimental.pallas import tpu_sc as plsc

NROWS = 4096
NCOLS = 4096
NNZ = 1638400
NW = 32
SPLITS = 2
CSPLIT = NCOLS // SPLITS
COLS_PER_W = CSPLIT // NW
SLAB = 8
SLABS = COLS_PER_W // SLAB
TILE = 2048
L = 16
UNROLL = 4
TBLK = 1024


def _make_sc_body(split):
    def body(ccol_hbm, ccols_hbm, rows_hbm, vals_hbm, outt_hbm, acc0, acc1,
             rbuf0, rbuf1, vbuf0, vbuf1, bbuf, bsbuf,
             sem0, sem1, fsem0, fsem1):
        return _sc_work(split, ccol_hbm, ccols_hbm, rows_hbm, vals_hbm,
                        outt_hbm, acc0, acc1, rbuf0, rbuf1, vbuf0, vbuf1,
                        bbuf, bsbuf, sem0, sem1, fsem0, fsem1)
    return body


def _sc_work(split, ccol_hbm, ccols_hbm, rows_hbm, vals_hbm, outt_hbm,
             acc0, acc1, rbuf0, rbuf1, vbuf0, vbuf1, bbuf, bsbuf,
             sem0, sem1, fsem0, fsem1):
    wid = lax.axis_index("s") * 2 + lax.axis_index("c")
    iota = jax.lax.iota(jnp.int32, L)
    zeros16 = jnp.zeros((L,), jnp.float32)
    accs = (acc0, acc1)
    rbufs = (rbuf0, rbuf1)
    vbufs = (vbuf0, vbuf1)
    sems = (sem0, sem1)
    fsems = (fsem0, fsem1)

    gw0 = split * CSPLIT + wid * COLS_PER_W
    cw0 = wid * COLS_PER_W
    pltpu.sync_copy(ccol_hbm.at[pl.ds(gw0, COLS_PER_W + L)], bbuf)
    pltpu.sync_copy(ccols_hbm.at[pl.ds(gw0, COLS_PER_W + L)], bsbuf)

    def run_slab(ci, half, first):
        acc = accs[half]
        fsem = fsems[half]
        c0 = cw0 + ci * SLAB
        b_lo = bbuf[pl.ds(ci * SLAB, L)]
        b_hi = bsbuf[pl.ds(ci * SLAB, L)]
        start = jnp.sum(jnp.where(iota == 0, b_lo, 0))
        end = jnp.sum(jnp.where(iota == SLAB - 1, b_hi, 0))

        if not first:
            pc0 = c0 - 2 * SLAB
            pltpu.make_async_copy(
                acc, outt_hbm.at[pl.ds(pc0, SLAB), :], fsem).wait()

        base0 = start & jnp.int32(-8)
        ntiles = (end - base0 + (TILE - 1)) // TILE

        def window(t):
            return pl.multiple_of(
                jnp.minimum(base0 + t * TILE, NNZ - TILE), 8)

        def issue(t, sub):
            tb = window(t)
            pltpu.async_copy(rows_hbm.at[pl.ds(tb, TILE)], rbufs[sub],
                             sems[sub])
            pltpu.async_copy(vals_hbm.at[pl.ds(tb, TILE)], vbufs[sub],
                             sems[sub])

        def wait(t, sub):
            tb = window(t)
            pltpu.make_async_copy(rows_hbm.at[pl.ds(tb, TILE)], rbufs[sub],
                                  sems[sub]).wait()
            pltpu.make_async_copy(vals_hbm.at[pl.ds(tb, TILE)], vbufs[sub],
                                  sems[sub]).wait()

        for sub in range(2):
            @pl.when(sub < ntiles)
            def _():
                issue(sub, sub)

        def zero_body(k, _):
            for j in range(SLAB):
                for u in range(2):
                    acc[j, pl.ds((k * 2 + u) * L, L)] = zeros16
            return 0

        lax.fori_loop(0, NROWS // L // 2, zero_body, 0)

        bcs = [jnp.sum(jnp.where(iota == j, b_hi, 0)) for j in range(SLAB)]

        def pair_body(p, _):
            for sub in range(2):
                t = p * 2 + sub

                @pl.when(t < ntiles)
                def _():
                    tb = window(t)
                    lo = jnp.maximum(start, base0 + t * TILE)
                    hi = jnp.minimum(end, base0 + (t + 1) * TILE)
                    wait(t, sub)

                    def vec_body(vi, _):
                        for u in range(UNROLL):
                            off = (vi * UNROLL + u) * L
                            g = tb + off + iota
                            r16 = rbufs[sub][pl.ds(off, L)]
                            vl = vbufs[sub][pl.ds(off, L)]
                            msk = (g >= lo) & (g < hi)
                            rel = jnp.zeros((L,), jnp.int32)
                            for bc in bcs:
                                rel = rel + jnp.where(g >= bc, 1, 0)
                            rel = jnp.minimum(rel, SLAB - 1)
                            plsc.addupdate_scatter(acc, [rel, r16], vl,
                                                   mask=msk)
                        return 0

                    lax.fori_loop(0, TILE // L // UNROLL, vec_body, 0)

                    @pl.when(t + 2 < ntiles)
                    def _():
                        issue(t + 2, sub)
            return 0

        lax.fori_loop(0, (ntiles + 1) // 2, pair_body, 0)

        pltpu.async_copy(acc, outt_hbm.at[pl.ds(c0, SLAB), :], fsem)

    def slab_pair(p, _):
        for half in range(2):
            run_slab(p * 2 + half, half, first=False)
        return 0

    for half in range(2):
        run_slab(half, half, first=True)
    lax.fori_loop(1, SLABS // 2, slab_pair, 0)

    for half in range(2):
        c0 = cw0 + (SLABS - 2 + half) * SLAB
        pltpu.make_async_copy(
            accs[half], outt_hbm.at[pl.ds(c0, SLAB), :], fsems[half]).wait()


def _transpose_body(int_ref, out_ref):
    out_ref[...] = int_ref[...].T


def _transpose_body_inplace(int_ref, din_ref, out_ref):
    del din_ref
    out_ref[...] = int_ref[...].T


def _run_sc_split(split, ccol_p, ccols_p, rows_p, vals_p):
    mesh = plsc.VectorSubcoreMesh(core_axis_name="c", subcore_axis_name="s")
    return pl.kernel(
        _make_sc_body(split),
        out_type=jax.ShapeDtypeStruct((CSPLIT, NROWS), jnp.float32),
        mesh=mesh,
        scratch_types=[
            pltpu.VMEM((SLAB, NROWS), jnp.float32),
            pltpu.VMEM((SLAB, NROWS), jnp.float32),
            pltpu.VMEM((TILE,), jnp.int32),
            pltpu.VMEM((TILE,), jnp.int32),
            pltpu.VMEM((TILE,), jnp.float32),
            pltpu.VMEM((TILE,), jnp.float32),
            pltpu.VMEM((COLS_PER_W + L,), jnp.int32),
            pltpu.VMEM((COLS_PER_W + L,), jnp.int32),
            pltpu.SemaphoreType.DMA,
            pltpu.SemaphoreType.DMA,
            pltpu.SemaphoreType.DMA,
            pltpu.SemaphoreType.DMA,
        ],
        compiler_params=pltpu.CompilerParams(needs_layout_passes=False),
    )(ccol_p, ccols_p, rows_p, vals_p)


def _run_tc_split(split, outt, dense_in):
    grid = (NROWS // TBLK, CSPLIT // TBLK)
    cofs = split * (CSPLIT // TBLK)
    if dense_in is None:
        return pl.pallas_call(
            _transpose_body,
            grid=grid,
            in_specs=[pl.BlockSpec((TBLK, TBLK), lambda i, j: (j, i))],
            out_specs=pl.BlockSpec((TBLK, TBLK),
                                   lambda i, j: (i, j + cofs)),
            out_shape=jax.ShapeDtypeStruct((NROWS, NCOLS), jnp.float32),
        )(outt)
    return pl.pallas_call(
        _transpose_body_inplace,
        grid=grid,
        in_specs=[
            pl.BlockSpec((TBLK, TBLK), lambda i, j: (j, i)),
            pl.BlockSpec(memory_space=pltpu.HBM),
        ],
        out_specs=pl.BlockSpec((TBLK, TBLK), lambda i, j: (i, j + cofs)),
        out_shape=jax.ShapeDtypeStruct((NROWS, NCOLS), jnp.float32),
        input_output_aliases={1: 0},
    )(outt, dense_in)


@jax.jit
def _csc_to_dense(ccol_p, ccols_p, rows_p, vals_p):
    outts = [_run_sc_split(s, ccol_p, ccols_p, rows_p, vals_p)
             for s in range(SPLITS)]
    dense = None
    for s in range(SPLITS):
        dense = _run_tc_split(s, outts[s], dense)
    return dense


def kernel(ccol_indices, row_indices, values):
    ccol = ccol_indices.astype(jnp.int32)
    rows = row_indices.astype(jnp.int32)
    vals = values.astype(jnp.float32)
    ccol_p = jnp.pad(ccol, (0, 4256 - ccol.shape[0]))
    ccols_p = jnp.pad(ccol[1:], (0, 4256 - ccol.shape[0] + 1))
    return _csc_to_dense(ccol_p, ccols_p, rows, vals)
